# Initial kernel scaffold; baseline (speedup 1.0000x reference)
import functools

import jax
import jax.numpy as jnp
from jax import lax
from jax.experimental import pallas as pl
from jax.experimental.pallas import tpu as pltpu
from jax.experimental.pallas import tpu_sc as plsc

N = 10000
E = 320000
D = 128
NSUB = 16
NCORE = 2
CH = 128
E_PAD = 323584
PAD = E_PAD - E
NP = 10240
NPS = NP // NSUB
BN = 2000


def _make_segsum(ec):
    ec_per = ec // NSUB
    n_chunks = ec_per // CH
    mesh = plsc.VectorSubcoreMesh(core_axis_name="c", subcore_axis_name="s")

    @functools.partial(
        pl.kernel,
        out_type=jax.ShapeDtypeStruct((2 * NP, D), jnp.float32),
        mesh=mesh,
        scratch_types=[
            pltpu.VMEM((ec_per,), jnp.int32),
            pltpu.VMEM((n_chunks, CH), jnp.int32),
            pltpu.VMEM((CH, D), jnp.float32),
            pltpu.VMEM_SHARED((NP, D), jnp.float32),
        ],
    )
    def seg(table, gidx, sidx2, zeros, out, gi_v, si_v, rows_v, acc):
        c = lax.axis_index("c")
        s = lax.axis_index("s")
        pltpu.sync_copy(zeros, acc.at[pl.ds(s * NPS, NPS)])
        pltpu.sync_copy(gidx.at[pl.ds(c * ec + s * ec_per, ec_per)], gi_v)
        pltpu.sync_copy(
            sidx2.at[pl.ds((c * NSUB + s) * n_chunks, n_chunks)], si_v)
        plsc.subcore_barrier()

        @pl.loop(0, n_chunks)
        def _(i):
            pltpu.sync_copy(table.at[gi_v.at[pl.ds(i * CH, CH)]], rows_v)
            pltpu.sync_copy(rows_v, acc.at[si_v.at[i]], add=True)

        plsc.subcore_barrier()
        pltpu.sync_copy(acc.at[pl.ds(s * NPS, NPS)],
                        out.at[pl.ds(c * NP + s * NPS, NPS)])

    return seg


def _make_degree():
    ec_per = E_PAD // NSUB
    n_chunks = ec_per // CH
    mesh = plsc.VectorSubcoreMesh(core_axis_name="c", subcore_axis_name="s")

    @functools.partial(
        pl.kernel,
        out_type=jax.ShapeDtypeStruct((2 * NP, 16), jnp.float32),
        mesh=mesh,
        scratch_types=[
            pltpu.VMEM((n_chunks, CH), jnp.int32),
            pltpu.VMEM((CH, 16), jnp.float32),
            pltpu.VMEM_SHARED((NP, 16), jnp.float32),
        ],
    )
    def deg(cidx2, ones_hbm, zeros16, out, si_v, ones_v, acc):
        c = lax.axis_index("c")
        s = lax.axis_index("s")
        pltpu.sync_copy(zeros16, acc.at[pl.ds(s * NPS, NPS)])
        pltpu.sync_copy(ones_hbm, ones_v)
        pltpu.sync_copy(
            cidx2.at[pl.ds((c * NSUB + s) * n_chunks, n_chunks)], si_v)
        plsc.subcore_barrier()

        @pl.loop(0, n_chunks)
        def _(i):
            pltpu.sync_copy(ones_v, acc.at[si_v.at[i]], add=True)

        plsc.subcore_barrier()
        pltpu.sync_copy(acc.at[pl.ds(s * NPS, NPS)],
                        out.at[pl.ds(c * NP + s * NPS, NPS)])

    return deg


_SEG_A = _make_segsum(E_PAD // 2)
_SEG_C = _make_segsum(E_PAD)
_DEGREE = _make_degree()


def _enc_body(x, we, be, cnt, h_ref, ht_ref):
    h = jnp.maximum(
        jnp.dot(x[...], we[...], preferred_element_type=jnp.float32)
        + be[...], 0.0)
    dinv = lax.rsqrt(cnt[...][1][:, 0:1] + 1.0)
    h_ref[...] = h
    ht_ref[...] = h * dinv


def _conv_body(p, h, cnt, wc, bc, wg, bg, hn_ref, g_ref):
    pp = p[...]
    dinv = lax.rsqrt(cnt[...][1][:, 0:1] + 1.0)
    a = dinv * (pp[0] + pp[1]) + (dinv * dinv) * h[...]
    hn = jnp.maximum(
        jnp.dot(a, wc[...], preferred_element_type=jnp.float32) + bc[...], 0.0)
    hg = jnp.maximum(
        jnp.dot(a, wg[...], preferred_element_type=jnp.float32) + bg[...], 0.0)
    hn_ref[...] = hn
    g_ref[...] = jnp.stack([hg, hg * hg])


def _gate_body(t, g, h, hn, cnt, ho_ref, hto_ref):
    tt = t[...]
    gg = g[...]
    cc = cnt[...]
    outdeg = cc[0][:, 0:1]
    dinv = lax.rsqrt(cc[1][:, 0:1] + 1.0)
    invc = 1.0 / jnp.maximum(outdeg, 1.0)
    s = outdeg * gg[1] - 2.0 * gg[0] * tt[0] + tt[1]
    tau = jnp.tanh(s * invc)
    ho = h[...] + tau * (hn[...] - h[...])
    ho_ref[...] = ho
    hto_ref[...] = ho * dinv


def _dec_body(h, wd, bd, o_ref):
    o_ref[...] = jnp.maximum(
        jnp.dot(h[...], wd[...], preferred_element_type=jnp.float32)
        + bd[...], 0.0)


_ROW = pl.BlockSpec((BN, D), lambda i: (i, 0))
_MAT = pl.BlockSpec((D, D), lambda i: (0, 0))
_VEC = pl.BlockSpec((1, D), lambda i: (0, 0))
_CNT = pl.BlockSpec((2, BN, 16), lambda i: (0, i, 0))
_ROW2 = pl.BlockSpec((2, BN, D), lambda i: (0, i, 0))
_F = jax.ShapeDtypeStruct((N, D), jnp.float32)
_F2 = jax.ShapeDtypeStruct((2, N, D), jnp.float32)


def _enc(x, we, be, cnt2):
    return pl.pallas_call(
        _enc_body, grid=(N // BN,),
        in_specs=[_ROW, _MAT, _VEC, _CNT],
        out_specs=[_ROW, _ROW], out_shape=[_F, _F],
    )(x, we, be, cnt2)


def _conv(p, h, cnt2, wc, bc, wg, bg):
    return pl.pallas_call(
        _conv_body, grid=(N // BN,),
        in_specs=[_ROW2, _ROW, _CNT, _MAT, _VEC, _MAT, _VEC],
        out_specs=[_ROW, _ROW2], out_shape=[_F, _F2],
    )(p, h, cnt2, wc, bc, wg, bg)


def _gate(t, g, h, hn, cnt2):
    return pl.pallas_call(
        _gate_body, grid=(N // BN,),
        in_specs=[_ROW2, _ROW2, _ROW, _ROW, _CNT],
        out_specs=[_ROW, _ROW], out_shape=[_F, _F],
    )(t, g, h, hn, cnt2)


def _dec(h, wd, bd):
    return pl.pallas_call(
        _dec_body, grid=(N // BN,),
        in_specs=[_ROW, _MAT, _VEC],
        out_specs=_ROW, out_shape=_F,
    )(h, wd, bd)


def kernel(X, edge_index, W_enc, b_enc, W_conv, b_conv, W_gg, b_gg,
           W_dec, b_dec):
    src0 = edge_index[0]
    dst0 = edge_index[1]
    pad_junk = jnp.full((PAD,), N, jnp.int32)
    pad_zero = jnp.zeros((PAD,), jnp.int32)
    src_s = jnp.concatenate([src0, pad_junk])
    dst_s = jnp.concatenate([dst0, pad_junk])
    src_g = jnp.concatenate([src0, pad_zero])
    dst_g = jnp.concatenate([dst0, pad_zero])

    zeros_d = jnp.zeros((NPS, D), jnp.float32)
    zeros_16 = jnp.zeros((NPS, 16), jnp.float32)
    ones_16 = jnp.ones((CH, 16), jnp.float32)

    cidx2 = jnp.concatenate([src_s, dst_s]).reshape(-1, CH)
    cnt2 = _DEGREE(cidx2, ones_16, zeros_16).reshape(2, NP, 16)[:, :N]

    be = b_enc.reshape(1, D)
    bc = b_conv.reshape(1, D)
    bg = b_gg.reshape(1, D)
    bd = b_dec.reshape(1, D)

    H, Ht = _enc(X, W_enc, be, cnt2)

    sidx_a = dst_s.reshape(-1, CH)
    gidx_c = jnp.concatenate([dst_g, dst_g + N])
    sidx_c = jnp.concatenate([src_s, src_s]).reshape(-1, CH)

    for _ in range(4):
        Pf = _SEG_A(Ht, src_g, sidx_a, zeros_d)
        P = Pf.reshape(2, NP, D)[:, :N]
        Hn, G = _conv(P, H, cnt2, W_conv, bc, W_gg, bg)
        Tf = _SEG_C(G.reshape(2 * N, D), gidx_c, sidx_c, zeros_d)
        T = Tf.reshape(2, NP, D)[:, :N]
        H, Ht = _gate(T, G, H, Hn, cnt2)

    return _dec(H, W_dec, bd)


# speedup vs baseline: 5.0751x; 5.0751x over previous
import functools

import jax
import jax.numpy as jnp
from jax import lax
from jax.experimental import pallas as pl
from jax.experimental.pallas import tpu as pltpu
from jax.experimental.pallas import tpu_sc as plsc

N = 10000
E = 320000
D = 128
NSUB = 16
NCORE = 2
CH = 128
E_PAD = 327680
PAD = E_PAD - E
NP = 10240
NPS = NP // NSUB
BN = 2000


def _make_segsum(ec):
    ec_per = ec // NSUB
    n_chunks = ec_per // CH
    ph = 40
    n_phases = n_chunks // ph
    mesh = plsc.VectorSubcoreMesh(core_axis_name="c", subcore_axis_name="s")

    @functools.partial(
        pl.kernel,
        out_type=jax.ShapeDtypeStruct((2 * NP, D), jnp.float32),
        mesh=mesh,
        scratch_types=[
            pltpu.VMEM((ph * CH,), jnp.int32),
            pltpu.VMEM((ph, CH), jnp.int32),
            pltpu.VMEM((CH, D), jnp.float32),
            pltpu.VMEM_SHARED((NP, D), jnp.float32),
        ],
    )
    def seg(table, gidx, sidx2, zeros, out, gi_v, si_v, rows_v, acc):
        c = lax.axis_index("c")
        s = lax.axis_index("s")
        pltpu.sync_copy(zeros, acc.at[pl.ds(s * NPS, NPS)])
        plsc.subcore_barrier()

        @pl.loop(0, n_phases)
        def _(p):
            pltpu.sync_copy(
                gidx.at[pl.ds(c * ec + s * ec_per + p * (ph * CH), ph * CH)],
                gi_v)
            pltpu.sync_copy(
                sidx2.at[pl.ds((c * NSUB + s) * n_chunks + p * ph, ph)], si_v)

            @pl.loop(0, ph)
            def _(i):
                pltpu.sync_copy(table.at[gi_v.at[pl.ds(i * CH, CH)]], rows_v)
                pltpu.sync_copy(rows_v, acc.at[si_v.at[i]], add=True)

        plsc.subcore_barrier()
        pltpu.sync_copy(acc.at[pl.ds(s * NPS, NPS)],
                        out.at[pl.ds(c * NP + s * NPS, NPS)])

    return seg


def _make_degree():
    ec_per = E_PAD // NSUB
    n_chunks = ec_per // CH
    ph = 40
    n_phases = n_chunks // ph
    mesh = plsc.VectorSubcoreMesh(core_axis_name="c", subcore_axis_name="s")

    @functools.partial(
        pl.kernel,
        out_type=jax.ShapeDtypeStruct((2 * NP, D), jnp.float32),
        mesh=mesh,
        scratch_types=[
            pltpu.VMEM((ph, CH), jnp.int32),
            pltpu.VMEM((CH, D), jnp.float32),
            pltpu.VMEM_SHARED((NP, D), jnp.float32),
        ],
    )
    def deg(cidx2, ones_hbm, zeros_d, out, si_v, ones_v, acc):
        c = lax.axis_index("c")
        s = lax.axis_index("s")
        pltpu.sync_copy(zeros_d, acc.at[pl.ds(s * NPS, NPS)])
        pltpu.sync_copy(ones_hbm, ones_v)
        plsc.subcore_barrier()

        @pl.loop(0, n_phases)
        def _(p):
            pltpu.sync_copy(
                cidx2.at[pl.ds((c * NSUB + s) * n_chunks + p * ph, ph)], si_v)

            @pl.loop(0, ph)
            def _(i):
                pltpu.sync_copy(ones_v, acc.at[si_v.at[i]], add=True)

        plsc.subcore_barrier()
        pltpu.sync_copy(acc.at[pl.ds(s * NPS, NPS)],
                        out.at[pl.ds(c * NP + s * NPS, NPS)])

    return deg


_CACHE = {}


def _seg_a(*args):
    if "a" not in _CACHE:
        _CACHE["a"] = _make_segsum(E_PAD // 2)
    return _CACHE["a"](*args)


def _seg_c(*args):
    if "c" not in _CACHE:
        _CACHE["c"] = _make_segsum(E_PAD)
    return _CACHE["c"](*args)


def _degree(*args):
    if "d" not in _CACHE:
        _CACHE["d"] = _make_degree()
    return _CACHE["d"](*args)


def _enc_body(x, we, be, cnt, h_ref, ht_ref):
    h = jnp.maximum(
        jnp.dot(x[...], we[...], preferred_element_type=jnp.float32)
        + be[...], 0.0)
    dinv = lax.rsqrt(cnt[...][1][:, 0:1] + 1.0)
    h_ref[...] = h
    ht_ref[...] = h * dinv


def _conv_body(p, h, cnt, wc, bc, wg, bg, hn_ref, g_ref):
    pp = p[...]
    dinv = lax.rsqrt(cnt[...][1][:, 0:1] + 1.0)
    a = dinv * (pp[0] + pp[1]) + (dinv * dinv) * h[...]
    hn = jnp.maximum(
        jnp.dot(a, wc[...], preferred_element_type=jnp.float32) + bc[...], 0.0)
    hg = jnp.maximum(
        jnp.dot(a, wg[...], preferred_element_type=jnp.float32) + bg[...], 0.0)
    hn_ref[...] = hn
    g_ref[...] = jnp.stack([hg, hg * hg])


def _gate_body(t, g, h, hn, cnt, ho_ref, hto_ref):
    tt = t[...]
    gg = g[...]
    cc = cnt[...]
    outdeg = cc[0][:, 0:1]
    dinv = lax.rsqrt(cc[1][:, 0:1] + 1.0)
    invc = 1.0 / jnp.maximum(outdeg, 1.0)
    s = outdeg * gg[1] - 2.0 * gg[0] * tt[0] + tt[1]
    tau = jnp.tanh(s * invc)
    ho = h[...] + tau * (hn[...] - h[...])
    ho_ref[...] = ho
    hto_ref[...] = ho * dinv


def _dec_body(h, wd, bd, o_ref):
    o_ref[...] = jnp.maximum(
        jnp.dot(h[...], wd[...], preferred_element_type=jnp.float32)
        + bd[...], 0.0)


_ROW = pl.BlockSpec((BN, D), lambda i: (i, 0))
_MAT = pl.BlockSpec((D, D), lambda i: (0, 0))
_VEC = pl.BlockSpec((1, D), lambda i: (0, 0))
_CNT = pl.BlockSpec((2, BN, 16), lambda i: (0, i, 0))
_ROW2 = pl.BlockSpec((2, BN, D), lambda i: (0, i, 0))
_F = jax.ShapeDtypeStruct((N, D), jnp.float32)
_F2 = jax.ShapeDtypeStruct((2, N, D), jnp.float32)


def _enc(x, we, be, cnt2):
    return pl.pallas_call(
        _enc_body, grid=(N // BN,),
        in_specs=[_ROW, _MAT, _VEC, _CNT],
        out_specs=[_ROW, _ROW], out_shape=[_F, _F],
    )(x, we, be, cnt2)


def _conv(p, h, cnt2, wc, bc, wg, bg):
    return pl.pallas_call(
        _conv_body, grid=(N // BN,),
        in_specs=[_ROW2, _ROW, _CNT, _MAT, _VEC, _MAT, _VEC],
        out_specs=[_ROW, _ROW2], out_shape=[_F, _F2],
    )(p, h, cnt2, wc, bc, wg, bg)


def _gate(t, g, h, hn, cnt2):
    return pl.pallas_call(
        _gate_body, grid=(N // BN,),
        in_specs=[_ROW2, _ROW2, _ROW, _ROW, _CNT],
        out_specs=[_ROW, _ROW], out_shape=[_F, _F],
    )(t, g, h, hn, cnt2)


def _dec(h, wd, bd):
    return pl.pallas_call(
        _dec_body, grid=(N // BN,),
        in_specs=[_ROW, _MAT, _VEC],
        out_specs=_ROW, out_shape=_F,
    )(h, wd, bd)


def kernel(X, edge_index, W_enc, b_enc, W_conv, b_conv, W_gg, b_gg,
           W_dec, b_dec):
    src0 = edge_index[0]
    dst0 = edge_index[1]
    pad_junk = jnp.full((PAD,), N, jnp.int32)
    pad_zero = jnp.zeros((PAD,), jnp.int32)
    src_s = jnp.concatenate([src0, pad_junk])
    dst_s = jnp.concatenate([dst0, pad_junk])
    src_g = jnp.concatenate([src0, pad_zero])
    dst_g = jnp.concatenate([dst0, pad_zero])

    zeros_d = jnp.zeros((NPS, D), jnp.float32)
    ones_d = jnp.ones((CH, D), jnp.float32)

    cidx2 = jnp.concatenate([src_s, dst_s]).reshape(-1, CH)
    cnt2 = _degree(cidx2, ones_d, zeros_d).reshape(2, NP, D)[:, :N, :16]

    be = b_enc.reshape(1, D)
    bc = b_conv.reshape(1, D)
    bg = b_gg.reshape(1, D)
    bd = b_dec.reshape(1, D)

    H, Ht = _enc(X, W_enc, be, cnt2)

    sidx_a = dst_s.reshape(-1, CH)
    gidx_c = jnp.concatenate([dst_g, dst_g + N])
    sidx_c = jnp.concatenate([src_s, src_s]).reshape(-1, CH)

    for _ in range(4):
        Pf = _seg_a(Ht, src_g, sidx_a, zeros_d)
        P = Pf.reshape(2, NP, D)[:, :N]
        Hn, G = _conv(P, H, cnt2, W_conv, bc, W_gg, bg)
        Tf = _seg_c(G.reshape(2 * N, D), gidx_c, sidx_c, zeros_d)
        T = Tf.reshape(2, NP, D)[:, :N]
        H, Ht = _gate(T, G, H, Hn, cnt2)

    return _dec(H, W_dec, bd)


# speedup vs baseline: 5.2893x; 1.0422x over previous
import functools

import jax
import jax.numpy as jnp
from jax import lax
from jax.experimental import pallas as pl
from jax.experimental.pallas import tpu as pltpu
from jax.experimental.pallas import tpu_sc as plsc

N = 10000
E = 320000
D = 128
NSUB = 16
NCORE = 2
CH = 128
E_PAD = 327680
PAD = E_PAD - E
NP = 10240
NPS = NP // NSUB
BN = 2000


def _make_segsum(ec):
    ec_per = ec // NSUB
    n_chunks = ec_per // CH
    ph = 40
    n_phases = n_chunks // ph
    mesh = plsc.VectorSubcoreMesh(core_axis_name="c", subcore_axis_name="s")

    @functools.partial(
        pl.kernel,
        out_type=jax.ShapeDtypeStruct((2 * NP, D), jnp.float32),
        mesh=mesh,
        scratch_types=[
            pltpu.VMEM((ph * CH,), jnp.int32),
            pltpu.VMEM((ph, CH), jnp.int32),
            pltpu.VMEM((CH, D), jnp.float32),
            pltpu.VMEM((CH, D), jnp.float32),
            pltpu.VMEM_SHARED((NP, D), jnp.float32),
            pltpu.SemaphoreType.DMA,
            pltpu.SemaphoreType.DMA,
            pltpu.SemaphoreType.DMA,
            pltpu.SemaphoreType.DMA,
        ],
    )
    def seg(table, gidx, sidx2, zeros, out, gi_v, si_v, rows0, rows1, acc,
            sg0, sg1, ss0, ss1):
        c = lax.axis_index("c")
        s = lax.axis_index("s")
        rows = (rows0, rows1)
        sg = (sg0, sg1)
        ss = (ss0, ss1)
        pltpu.sync_copy(zeros, acc.at[pl.ds(s * NPS, NPS)])
        plsc.subcore_barrier()

        @pl.loop(0, n_phases)
        def _(p):
            pltpu.sync_copy(
                gidx.at[pl.ds(c * ec + s * ec_per + p * (ph * CH), ph * CH)],
                gi_v)
            pltpu.sync_copy(
                sidx2.at[pl.ds((c * NSUB + s) * n_chunks + p * ph, ph)], si_v)

            @pl.loop(0, ph // 2)
            def _(j):
                g = [pltpu.async_copy(
                        table.at[gi_v.at[pl.ds((2 * j + b) * CH, CH)]],
                        rows[b], sg[b]) for b in (0, 1)]
                sc = []
                for b in (0, 1):
                    g[b].wait()
                    sc.append(pltpu.async_copy(
                        rows[b], acc.at[si_v.at[2 * j + b]], ss[b], add=True))
                for b in (0, 1):
                    sc[b].wait()

        plsc.subcore_barrier()
        pltpu.sync_copy(acc.at[pl.ds(s * NPS, NPS)],
                        out.at[pl.ds(c * NP + s * NPS, NPS)])

    return seg


def _make_degree():
    ec_per = E_PAD // NSUB
    n_chunks = ec_per // CH
    ph = 40
    n_phases = n_chunks // ph
    mesh = plsc.VectorSubcoreMesh(core_axis_name="c", subcore_axis_name="s")

    @functools.partial(
        pl.kernel,
        out_type=jax.ShapeDtypeStruct((2 * NP, D), jnp.float32),
        mesh=mesh,
        scratch_types=[
            pltpu.VMEM((ph, CH), jnp.int32),
            pltpu.VMEM((CH, D), jnp.float32),
            pltpu.VMEM_SHARED((NP, D), jnp.float32),
            pltpu.SemaphoreType.DMA,
            pltpu.SemaphoreType.DMA,
        ],
    )
    def deg(cidx2, ones_hbm, zeros_d, out, si_v, ones_v, acc, ss0, ss1):
        c = lax.axis_index("c")
        s = lax.axis_index("s")
        ss = (ss0, ss1)
        pltpu.sync_copy(zeros_d, acc.at[pl.ds(s * NPS, NPS)])
        pltpu.sync_copy(ones_hbm, ones_v)
        plsc.subcore_barrier()

        @pl.loop(0, n_phases)
        def _(p):
            pltpu.sync_copy(
                cidx2.at[pl.ds((c * NSUB + s) * n_chunks + p * ph, ph)], si_v)

            @pl.loop(0, ph // 2)
            def _(j):
                sc = [pltpu.async_copy(
                        ones_v, acc.at[si_v.at[2 * j + b]], ss[b], add=True)
                      for b in (0, 1)]
                for b in (0, 1):
                    sc[b].wait()

        plsc.subcore_barrier()
        pltpu.sync_copy(acc.at[pl.ds(s * NPS, NPS)],
                        out.at[pl.ds(c * NP + s * NPS, NPS)])

    return deg


_CACHE = {}


def _seg_a(*args):
    if "a" not in _CACHE:
        _CACHE["a"] = _make_segsum(E_PAD // 2)
    return _CACHE["a"](*args)


def _seg_c(*args):
    if "c" not in _CACHE:
        _CACHE["c"] = _make_segsum(E_PAD)
    return _CACHE["c"](*args)


def _degree(*args):
    if "d" not in _CACHE:
        _CACHE["d"] = _make_degree()
    return _CACHE["d"](*args)


def _enc_body(x, we, be, cnt, h_ref, ht_ref):
    h = jnp.maximum(
        jnp.dot(x[...], we[...], preferred_element_type=jnp.float32)
        + be[...], 0.0)
    dinv = lax.rsqrt(cnt[...][1][:, 0:1] + 1.0)
    h_ref[...] = h
    ht_ref[...] = h * dinv


def _conv_body(p, h, cnt, wc, bc, wg, bg, hn_ref, g_ref):
    pp = p[...]
    dinv = lax.rsqrt(cnt[...][1][:, 0:1] + 1.0)
    a = dinv * (pp[0] + pp[1]) + (dinv * dinv) * h[...]
    hn = jnp.maximum(
        jnp.dot(a, wc[...], preferred_element_type=jnp.float32) + bc[...], 0.0)
    hg = jnp.maximum(
        jnp.dot(a, wg[...], preferred_element_type=jnp.float32) + bg[...], 0.0)
    hn_ref[...] = hn
    g_ref[...] = jnp.stack([hg, hg * hg])


def _gate_body(t, g, h, hn, cnt, ho_ref, hto_ref):
    tt = t[...]
    gg = g[...]
    cc = cnt[...]
    outdeg = cc[0][:, 0:1]
    dinv = lax.rsqrt(cc[1][:, 0:1] + 1.0)
    invc = 1.0 / jnp.maximum(outdeg, 1.0)
    s = outdeg * gg[1] - 2.0 * gg[0] * tt[0] + tt[1]
    tau = jnp.tanh(s * invc)
    ho = h[...] + tau * (hn[...] - h[...])
    ho_ref[...] = ho
    hto_ref[...] = ho * dinv


def _dec_body(h, wd, bd, o_ref):
    o_ref[...] = jnp.maximum(
        jnp.dot(h[...], wd[...], preferred_element_type=jnp.float32)
        + bd[...], 0.0)


_ROW = pl.BlockSpec((BN, D), lambda i: (i, 0))
_MAT = pl.BlockSpec((D, D), lambda i: (0, 0))
_VEC = pl.BlockSpec((1, D), lambda i: (0, 0))
_CNT = pl.BlockSpec((2, BN, 16), lambda i: (0, i, 0))
_ROW2 = pl.BlockSpec((2, BN, D), lambda i: (0, i, 0))
_F = jax.ShapeDtypeStruct((N, D), jnp.float32)
_F2 = jax.ShapeDtypeStruct((2, N, D), jnp.float32)


def _enc(x, we, be, cnt2):
    return pl.pallas_call(
        _enc_body, grid=(N // BN,),
        in_specs=[_ROW, _MAT, _VEC, _CNT],
        out_specs=[_ROW, _ROW], out_shape=[_F, _F],
    )(x, we, be, cnt2)


def _conv(p, h, cnt2, wc, bc, wg, bg):
    return pl.pallas_call(
        _conv_body, grid=(N // BN,),
        in_specs=[_ROW2, _ROW, _CNT, _MAT, _VEC, _MAT, _VEC],
        out_specs=[_ROW, _ROW2], out_shape=[_F, _F2],
    )(p, h, cnt2, wc, bc, wg, bg)


def _gate(t, g, h, hn, cnt2):
    return pl.pallas_call(
        _gate_body, grid=(N // BN,),
        in_specs=[_ROW2, _ROW2, _ROW, _ROW, _CNT],
        out_specs=[_ROW, _ROW], out_shape=[_F, _F],
    )(t, g, h, hn, cnt2)


def _dec(h, wd, bd):
    return pl.pallas_call(
        _dec_body, grid=(N // BN,),
        in_specs=[_ROW, _MAT, _VEC],
        out_specs=_ROW, out_shape=_F,
    )(h, wd, bd)


def kernel(X, edge_index, W_enc, b_enc, W_conv, b_conv, W_gg, b_gg,
           W_dec, b_dec):
    src0 = edge_index[0]
    dst0 = edge_index[1]
    pad_junk = jnp.full((PAD,), N, jnp.int32)
    pad_zero = jnp.zeros((PAD,), jnp.int32)
    src_s = jnp.concatenate([src0, pad_junk])
    dst_s = jnp.concatenate([dst0, pad_junk])
    src_g = jnp.concatenate([src0, pad_zero])
    dst_g = jnp.concatenate([dst0, pad_zero])

    zeros_d = jnp.zeros((NPS, D), jnp.float32)
    ones_d = jnp.ones((CH, D), jnp.float32)

    cidx2 = jnp.concatenate([src_s, dst_s]).reshape(-1, CH)
    cnt2 = _degree(cidx2, ones_d, zeros_d).reshape(2, NP, D)[:, :N, :16]

    be = b_enc.reshape(1, D)
    bc = b_conv.reshape(1, D)
    bg = b_gg.reshape(1, D)
    bd = b_dec.reshape(1, D)

    H, Ht = _enc(X, W_enc, be, cnt2)

    sidx_a = dst_s.reshape(-1, CH)
    gidx_c = jnp.concatenate([dst_g, dst_g + N])
    sidx_c = jnp.concatenate([src_s, src_s]).reshape(-1, CH)

    for _ in range(4):
        Pf = _seg_a(Ht, src_g, sidx_a, zeros_d)
        P = Pf.reshape(2, NP, D)[:, :N]
        Hn, G = _conv(P, H, cnt2, W_conv, bc, W_gg, bg)
        Tf = _seg_c(G.reshape(2 * N, D), gidx_c, sidx_c, zeros_d)
        T = Tf.reshape(2, NP, D)[:, :N]
        H, Ht = _gate(T, G, H, Hn, cnt2)

    return _dec(H, W_dec, bd)


# speedup vs baseline: 5.8585x; 1.1076x over previous
import functools

import jax
import jax.numpy as jnp
from jax import lax
from jax.experimental import pallas as pl
from jax.experimental.pallas import tpu as pltpu
from jax.experimental.pallas import tpu_sc as plsc

N = 10000
E = 320000
D = 128
NSUB = 16
NCORE = 2
CH = 64
E_PAD = 327680
PAD = E_PAD - E
NP = 10240
NPS = NP // NSUB
BN = 2000
NSLOT = 4
PH = 16


def _make_segsum(ec):
    ec_per = ec // NSUB
    n_chunks = ec_per // CH
    n_phases = n_chunks // PH
    mesh = plsc.VectorSubcoreMesh(core_axis_name="c", subcore_axis_name="s")

    @functools.partial(
        pl.kernel,
        out_type=jax.ShapeDtypeStruct((2 * NP, D), jnp.float32),
        mesh=mesh,
        scratch_types=[
            pltpu.VMEM((PH * CH,), jnp.int32),
            pltpu.VMEM((PH, CH), jnp.int32),
            pltpu.VMEM((NSLOT, CH, D), jnp.float32),
            pltpu.VMEM_SHARED((NP, D), jnp.float32),
        ] + [pltpu.SemaphoreType.DMA] * (2 * NSLOT),
    )
    def seg(table, gidx, sidx2, zeros, out, gi_v, si_v, rows, acc, *sems):
        c = lax.axis_index("c")
        s = lax.axis_index("s")
        sg = sems[:NSLOT]
        ss = sems[NSLOT:]
        pltpu.sync_copy(zeros, acc.at[pl.ds(s * NPS, NPS)])
        plsc.subcore_barrier()

        @pl.loop(0, n_phases)
        def _(p):
            pltpu.sync_copy(
                gidx.at[pl.ds(c * ec + s * ec_per + p * (PH * CH), PH * CH)],
                gi_v)
            pltpu.sync_copy(
                sidx2.at[pl.ds((c * NSUB + s) * n_chunks + p * PH, PH)], si_v)

            hg = [None] * NSLOT
            hs = [None] * NSLOT
            for i in range(PH):
                b = i % NSLOT
                if hs[b] is not None:
                    hs[b].wait()
                hg[b] = pltpu.async_copy(
                    table.at[gi_v.at[pl.ds(i * CH, CH)]], rows.at[b], sg[b])
                if i >= 2:
                    b2 = (i - 2) % NSLOT
                    hg[b2].wait()
                    hs[b2] = pltpu.async_copy(
                        rows.at[b2], acc.at[si_v.at[i - 2]], ss[b2], add=True)
            for i in (PH - 2, PH - 1):
                b = i % NSLOT
                hg[b].wait()
                hs[b] = pltpu.async_copy(
                    rows.at[b], acc.at[si_v.at[i]], ss[b], add=True)
            for b in range(NSLOT):
                if hs[b] is not None:
                    hs[b].wait()

        plsc.subcore_barrier()
        pltpu.sync_copy(acc.at[pl.ds(s * NPS, NPS)],
                        out.at[pl.ds(c * NP + s * NPS, NPS)])

    return seg


def _make_degree():
    ec_per = E_PAD // NSUB
    n_chunks = ec_per // CH
    n_phases = n_chunks // PH
    mesh = plsc.VectorSubcoreMesh(core_axis_name="c", subcore_axis_name="s")

    @functools.partial(
        pl.kernel,
        out_type=jax.ShapeDtypeStruct((2 * NP, D), jnp.float32),
        mesh=mesh,
        scratch_types=[
            pltpu.VMEM((PH, CH), jnp.int32),
            pltpu.VMEM((CH, D), jnp.float32),
            pltpu.VMEM_SHARED((NP, D), jnp.float32),
        ] + [pltpu.SemaphoreType.DMA] * NSLOT,
    )
    def deg(cidx2, ones_hbm, zeros_d, out, si_v, ones_v, acc, *ss):
        c = lax.axis_index("c")
        s = lax.axis_index("s")
        pltpu.sync_copy(zeros_d, acc.at[pl.ds(s * NPS, NPS)])
        pltpu.sync_copy(ones_hbm, ones_v)
        plsc.subcore_barrier()

        @pl.loop(0, n_phases)
        def _(p):
            pltpu.sync_copy(
                cidx2.at[pl.ds((c * NSUB + s) * n_chunks + p * PH, PH)], si_v)

            hs = [None] * NSLOT
            for i in range(PH):
                b = i % NSLOT
                if hs[b] is not None:
                    hs[b].wait()
                hs[b] = pltpu.async_copy(
                    ones_v, acc.at[si_v.at[i]], ss[b], add=True)
            for b in range(NSLOT):
                if hs[b] is not None:
                    hs[b].wait()

        plsc.subcore_barrier()
        pltpu.sync_copy(acc.at[pl.ds(s * NPS, NPS)],
                        out.at[pl.ds(c * NP + s * NPS, NPS)])

    return deg


_CACHE = {}


def _seg_a(*args):
    if "a" not in _CACHE:
        _CACHE["a"] = _make_segsum(E_PAD // 2)
    return _CACHE["a"](*args)


def _seg_c(*args):
    if "c" not in _CACHE:
        _CACHE["c"] = _make_segsum(E_PAD)
    return _CACHE["c"](*args)


def _degree(*args):
    if "d" not in _CACHE:
        _CACHE["d"] = _make_degree()
    return _CACHE["d"](*args)


def _enc_body(x, we, be, cnt, h_ref, ht_ref):
    h = jnp.maximum(
        jnp.dot(x[...], we[...], preferred_element_type=jnp.float32)
        + be[...], 0.0)
    dinv = lax.rsqrt(cnt[...][1][:, 0:1] + 1.0)
    h_ref[...] = h
    ht_ref[...] = h * dinv


def _conv_body(p, h, cnt, wc, bc, wg, bg, hn_ref, g_ref):
    pp = p[...]
    dinv = lax.rsqrt(cnt[...][1][:, 0:1] + 1.0)
    a = dinv * (pp[0] + pp[1]) + (dinv * dinv) * h[...]
    hn = jnp.maximum(
        jnp.dot(a, wc[...], preferred_element_type=jnp.float32) + bc[...], 0.0)
    hg = jnp.maximum(
        jnp.dot(a, wg[...], preferred_element_type=jnp.float32) + bg[...], 0.0)
    hn_ref[...] = hn
    g_ref[...] = jnp.stack([hg, hg * hg])


def _gate_body(t, g, h, hn, cnt, ho_ref, hto_ref):
    tt = t[...]
    gg = g[...]
    cc = cnt[...]
    outdeg = cc[0][:, 0:1]
    dinv = lax.rsqrt(cc[1][:, 0:1] + 1.0)
    invc = 1.0 / jnp.maximum(outdeg, 1.0)
    s = outdeg * gg[1] - 2.0 * gg[0] * tt[0] + tt[1]
    tau = jnp.tanh(s * invc)
    ho = h[...] + tau * (hn[...] - h[...])
    ho_ref[...] = ho
    hto_ref[...] = ho * dinv


def _dec_body(h, wd, bd, o_ref):
    o_ref[...] = jnp.maximum(
        jnp.dot(h[...], wd[...], preferred_element_type=jnp.float32)
        + bd[...], 0.0)


_ROW = pl.BlockSpec((BN, D), lambda i: (i, 0))
_MAT = pl.BlockSpec((D, D), lambda i: (0, 0))
_VEC = pl.BlockSpec((1, D), lambda i: (0, 0))
_CNT = pl.BlockSpec((2, BN, 16), lambda i: (0, i, 0))
_ROW2 = pl.BlockSpec((2, BN, D), lambda i: (0, i, 0))
_F = jax.ShapeDtypeStruct((N, D), jnp.float32)
_F2 = jax.ShapeDtypeStruct((2, N, D), jnp.float32)


def _enc(x, we, be, cnt2):
    return pl.pallas_call(
        _enc_body, grid=(N // BN,),
        in_specs=[_ROW, _MAT, _VEC, _CNT],
        out_specs=[_ROW, _ROW], out_shape=[_F, _F],
    )(x, we, be, cnt2)


def _conv(p, h, cnt2, wc, bc, wg, bg):
    return pl.pallas_call(
        _conv_body, grid=(N // BN,),
        in_specs=[_ROW2, _ROW, _CNT, _MAT, _VEC, _MAT, _VEC],
        out_specs=[_ROW, _ROW2], out_shape=[_F, _F2],
    )(p, h, cnt2, wc, bc, wg, bg)


def _gate(t, g, h, hn, cnt2):
    return pl.pallas_call(
        _gate_body, grid=(N // BN,),
        in_specs=[_ROW2, _ROW2, _ROW, _ROW, _CNT],
        out_specs=[_ROW, _ROW], out_shape=[_F, _F],
    )(t, g, h, hn, cnt2)


def _dec(h, wd, bd):
    return pl.pallas_call(
        _dec_body, grid=(N // BN,),
        in_specs=[_ROW, _MAT, _VEC],
        out_specs=_ROW, out_shape=_F,
    )(h, wd, bd)


def kernel(X, edge_index, W_enc, b_enc, W_conv, b_conv, W_gg, b_gg,
           W_dec, b_dec):
    src0 = edge_index[0]
    dst0 = edge_index[1]
    pad_junk = jnp.full((PAD,), N, jnp.int32)
    pad_zero = jnp.zeros((PAD,), jnp.int32)
    src_s = jnp.concatenate([src0, pad_junk])
    dst_s = jnp.concatenate([dst0, pad_junk])
    src_g = jnp.concatenate([src0, pad_zero])
    dst_g = jnp.concatenate([dst0, pad_zero])

    zeros_d = jnp.zeros((NPS, D), jnp.float32)
    ones_d = jnp.ones((CH, D), jnp.float32)

    cidx2 = jnp.concatenate([src_s, dst_s]).reshape(-1, CH)
    cnt2 = _degree(cidx2, ones_d, zeros_d).reshape(2, NP, D)[:, :N, :16]

    be = b_enc.reshape(1, D)
    bc = b_conv.reshape(1, D)
    bg = b_gg.reshape(1, D)
    bd = b_dec.reshape(1, D)

    H, Ht = _enc(X, W_enc, be, cnt2)

    sidx_a = dst_s.reshape(-1, CH)
    gidx_c = jnp.concatenate([dst_g, dst_g + N])
    sidx_c = jnp.concatenate([src_s, src_s]).reshape(-1, CH)

    for _ in range(4):
        Pf = _seg_a(Ht, src_g, sidx_a, zeros_d)
        P = Pf.reshape(2, NP, D)[:, :N]
        Hn, G = _conv(P, H, cnt2, W_conv, bc, W_gg, bg)
        Tf = _seg_c(G.reshape(2 * N, D), gidx_c, sidx_c, zeros_d)
        T = Tf.reshape(2, NP, D)[:, :N]
        H, Ht = _gate(T, G, H, Hn, cnt2)

    return _dec(H, W_dec, bd)


# speedup vs baseline: 5.8894x; 1.0053x over previous
import functools

import jax
import jax.numpy as jnp
from jax import lax
from jax.experimental import pallas as pl
from jax.experimental.pallas import tpu as pltpu
from jax.experimental.pallas import tpu_sc as plsc

N = 10000
E = 320000
D = 128
NSUB = 16
NCORE = 2
CH = 64
E_PAD = 327680
PAD = E_PAD - E
NP = 10240
NPS = NP // NSUB
BN = 2000
NSLOT = 5
PH = 16


def _make_segsum(ec):
    ec_per = ec // NSUB
    n_chunks = ec_per // CH
    n_phases = n_chunks // PH
    mesh = plsc.VectorSubcoreMesh(core_axis_name="c", subcore_axis_name="s")

    @functools.partial(
        pl.kernel,
        out_type=jax.ShapeDtypeStruct((2 * NP, D), jnp.float32),
        mesh=mesh,
        scratch_types=[
            pltpu.VMEM((PH * CH,), jnp.int32),
            pltpu.VMEM((PH, CH), jnp.int32),
            pltpu.VMEM((NSLOT, CH, D), jnp.float32),
            pltpu.VMEM_SHARED((NP, D), jnp.float32),
        ] + [pltpu.SemaphoreType.DMA] * (2 * NSLOT),
    )
    def seg(table, gidx, sidx2, zeros, out, gi_v, si_v, rows, acc, *sems):
        c = lax.axis_index("c")
        s = lax.axis_index("s")
        sg = sems[:NSLOT]
        ss = sems[NSLOT:]
        pltpu.sync_copy(zeros, acc.at[pl.ds(s * NPS, NPS)])
        plsc.subcore_barrier()

        @pl.loop(0, n_phases)
        def _(p):
            pltpu.sync_copy(
                gidx.at[pl.ds(c * ec + s * ec_per + p * (PH * CH), PH * CH)],
                gi_v)
            pltpu.sync_copy(
                sidx2.at[pl.ds((c * NSUB + s) * n_chunks + p * PH, PH)], si_v)

            hg = [None] * NSLOT
            hs = [None] * NSLOT
            for i in range(PH):
                b = i % NSLOT
                if hs[b] is not None:
                    hs[b].wait()
                hg[b] = pltpu.async_copy(
                    table.at[gi_v.at[pl.ds(i * CH, CH)]], rows.at[b], sg[b])
                if i >= 3:
                    b2 = (i - 3) % NSLOT
                    hg[b2].wait()
                    hs[b2] = pltpu.async_copy(
                        rows.at[b2], acc.at[si_v.at[i - 3]], ss[b2], add=True)
            for i in (PH - 3, PH - 2, PH - 1):
                b = i % NSLOT
                hg[b].wait()
                hs[b] = pltpu.async_copy(
                    rows.at[b], acc.at[si_v.at[i]], ss[b], add=True)
            for b in range(NSLOT):
                if hs[b] is not None:
                    hs[b].wait()

        plsc.subcore_barrier()
        pltpu.sync_copy(acc.at[pl.ds(s * NPS, NPS)],
                        out.at[pl.ds(c * NP + s * NPS, NPS)])

    return seg


DW = 128


def _make_degree():
    ec_per = E_PAD // NSUB
    n_chunks = ec_per // CH
    n_phases = n_chunks // PH
    mesh = plsc.VectorSubcoreMesh(core_axis_name="c", subcore_axis_name="s")

    @functools.partial(
        pl.kernel,
        out_type=jax.ShapeDtypeStruct((2 * NP, DW), jnp.float32),
        mesh=mesh,
        scratch_types=[
            pltpu.VMEM((PH, CH), jnp.int32),
            pltpu.VMEM((CH, DW), jnp.float32),
            pltpu.VMEM_SHARED((NP, DW), jnp.float32),
        ] + [pltpu.SemaphoreType.DMA] * NSLOT,
    )
    def deg(cidx2, ones_hbm, zeros_d, out, si_v, ones_v, acc, *ss):
        c = lax.axis_index("c")
        s = lax.axis_index("s")
        pltpu.sync_copy(zeros_d, acc.at[pl.ds(s * NPS, NPS)])
        pltpu.sync_copy(ones_hbm, ones_v)
        plsc.subcore_barrier()

        @pl.loop(0, n_phases)
        def _(p):
            pltpu.sync_copy(
                cidx2.at[pl.ds((c * NSUB + s) * n_chunks + p * PH, PH)], si_v)

            hs = [None] * NSLOT
            for i in range(PH):
                b = i % NSLOT
                if hs[b] is not None:
                    hs[b].wait()
                hs[b] = pltpu.async_copy(
                    ones_v, acc.at[si_v.at[i]], ss[b], add=True)
            for b in range(NSLOT):
                if hs[b] is not None:
                    hs[b].wait()

        plsc.subcore_barrier()
        pltpu.sync_copy(acc.at[pl.ds(s * NPS, NPS)],
                        out.at[pl.ds(c * NP + s * NPS, NPS)])

    return deg


_CACHE = {}


def _seg_a(*args):
    if "a" not in _CACHE:
        _CACHE["a"] = _make_segsum(E_PAD // 2)
    return _CACHE["a"](*args)


def _seg_c(*args):
    if "c" not in _CACHE:
        _CACHE["c"] = _make_segsum(E_PAD)
    return _CACHE["c"](*args)


def _degree(*args):
    if "d" not in _CACHE:
        _CACHE["d"] = _make_degree()
    return _CACHE["d"](*args)


def _enc_body(x, we, be, cnt, h_ref, ht_ref):
    h = jnp.maximum(
        jnp.dot(x[...], we[...], preferred_element_type=jnp.float32)
        + be[...], 0.0)
    dinv = lax.rsqrt(cnt[...][1][:, 0:1] + 1.0)
    h_ref[...] = h
    ht_ref[...] = h * dinv


def _conv_body(p, h, cnt, wc, bc, wg, bg, hn_ref, g_ref):
    pp = p[...]
    dinv = lax.rsqrt(cnt[...][1][:, 0:1] + 1.0)
    a = dinv * (pp[0] + pp[1]) + (dinv * dinv) * h[...]
    hn = jnp.maximum(
        jnp.dot(a, wc[...], preferred_element_type=jnp.float32) + bc[...], 0.0)
    hg = jnp.maximum(
        jnp.dot(a, wg[...], preferred_element_type=jnp.float32) + bg[...], 0.0)
    hn_ref[...] = hn
    g_ref[...] = jnp.stack([hg, hg * hg])


def _gate_body(t, g, h, hn, cnt, ho_ref, hto_ref):
    tt = t[...]
    gg = g[...]
    cc = cnt[...]
    outdeg = cc[0][:, 0:1]
    dinv = lax.rsqrt(cc[1][:, 0:1] + 1.0)
    invc = 1.0 / jnp.maximum(outdeg, 1.0)
    s = outdeg * gg[1] - 2.0 * gg[0] * tt[0] + tt[1]
    tau = jnp.tanh(s * invc)
    ho = h[...] + tau * (hn[...] - h[...])
    ho_ref[...] = ho
    hto_ref[...] = ho * dinv


def _dec_body(h, wd, bd, o_ref):
    o_ref[...] = jnp.maximum(
        jnp.dot(h[...], wd[...], preferred_element_type=jnp.float32)
        + bd[...], 0.0)


_ROW = pl.BlockSpec((BN, D), lambda i: (i, 0))
_MAT = pl.BlockSpec((D, D), lambda i: (0, 0))
_VEC = pl.BlockSpec((1, D), lambda i: (0, 0))
_CNT = pl.BlockSpec((2, BN, 16), lambda i: (0, i, 0))
_ROW2 = pl.BlockSpec((2, BN, D), lambda i: (0, i, 0))
_F = jax.ShapeDtypeStruct((N, D), jnp.float32)
_F2 = jax.ShapeDtypeStruct((2, N, D), jnp.float32)


def _enc(x, we, be, cnt2):
    return pl.pallas_call(
        _enc_body, grid=(N // BN,),
        in_specs=[_ROW, _MAT, _VEC, _CNT],
        out_specs=[_ROW, _ROW], out_shape=[_F, _F],
    )(x, we, be, cnt2)


def _conv(p, h, cnt2, wc, bc, wg, bg):
    return pl.pallas_call(
        _conv_body, grid=(N // BN,),
        in_specs=[_ROW2, _ROW, _CNT, _MAT, _VEC, _MAT, _VEC],
        out_specs=[_ROW, _ROW2], out_shape=[_F, _F2],
    )(p, h, cnt2, wc, bc, wg, bg)


def _gate(t, g, h, hn, cnt2):
    return pl.pallas_call(
        _gate_body, grid=(N // BN,),
        in_specs=[_ROW2, _ROW2, _ROW, _ROW, _CNT],
        out_specs=[_ROW, _ROW], out_shape=[_F, _F],
    )(t, g, h, hn, cnt2)


def _dec(h, wd, bd):
    return pl.pallas_call(
        _dec_body, grid=(N // BN,),
        in_specs=[_ROW, _MAT, _VEC],
        out_specs=_ROW, out_shape=_F,
    )(h, wd, bd)


def kernel(X, edge_index, W_enc, b_enc, W_conv, b_conv, W_gg, b_gg,
           W_dec, b_dec):
    src0 = edge_index[0]
    dst0 = edge_index[1]
    pad_junk = jnp.full((PAD,), N, jnp.int32)
    pad_zero = jnp.zeros((PAD,), jnp.int32)
    src_s = jnp.concatenate([src0, pad_junk])
    dst_s = jnp.concatenate([dst0, pad_junk])
    src_g = jnp.concatenate([src0, pad_zero])
    dst_g = jnp.concatenate([dst0, pad_zero])

    zeros_d = jnp.zeros((NPS, D), jnp.float32)
    zeros_w = jnp.zeros((NPS, DW), jnp.float32)
    ones_w = jnp.ones((CH, DW), jnp.float32)

    cidx2 = jnp.concatenate([src_s, dst_s]).reshape(-1, CH)
    cnt2 = _degree(cidx2, ones_w, zeros_w).reshape(2, NP, DW)[:, :N, :16]

    be = b_enc.reshape(1, D)
    bc = b_conv.reshape(1, D)
    bg = b_gg.reshape(1, D)
    bd = b_dec.reshape(1, D)

    H, Ht = _enc(X, W_enc, be, cnt2)

    sidx_a = dst_s.reshape(-1, CH)
    gidx_c = jnp.concatenate([dst_g, dst_g + N])
    sidx_c = jnp.concatenate([src_s, src_s]).reshape(-1, CH)

    for _ in range(4):
        Pf = _seg_a(Ht, src_g, sidx_a, zeros_d)
        P = Pf.reshape(2, NP, D)[:, :N]
        Hn, G = _conv(P, H, cnt2, W_conv, bc, W_gg, bg)
        Tf = _seg_c(G.reshape(2 * N, D), gidx_c, sidx_c, zeros_d)
        T = Tf.reshape(2, NP, D)[:, :N]
        H, Ht = _gate(T, G, H, Hn, cnt2)

    return _dec(H, W_dec, bd)


# speedup vs baseline: 6.0515x; 1.0275x over previous
import functools

import jax
import jax.numpy as jnp
from jax import lax
from jax.experimental import pallas as pl
from jax.experimental.pallas import tpu as pltpu
from jax.experimental.pallas import tpu_sc as plsc

N = 10000
E = 320000
D = 128
NSUB = 16
NCORE = 2
CH = 64
E_PAD = 327680
PAD = E_PAD - E
NP = 10240
NPS = NP // NSUB
BN = 2000
NSLOT = 5
PH = 16


def _make_segsum(ec):
    ec_per = ec // NSUB
    n_chunks = ec_per // CH
    n_phases = n_chunks // PH
    mesh = plsc.VectorSubcoreMesh(core_axis_name="c", subcore_axis_name="s")

    @functools.partial(
        pl.kernel,
        out_type=jax.ShapeDtypeStruct((2 * NP, D), jnp.float32),
        mesh=mesh,
        scratch_types=[
            pltpu.VMEM((PH * CH,), jnp.int32),
            pltpu.VMEM((PH * CH,), jnp.int32),
            pltpu.VMEM((PH, CH), jnp.int32),
            pltpu.VMEM((PH, CH), jnp.int32),
            pltpu.VMEM((NSLOT, CH, D), jnp.float32),
            pltpu.VMEM_SHARED((NP, D), jnp.float32),
        ] + [pltpu.SemaphoreType.DMA] * (2 * NSLOT + 4),
    )
    def seg(table, gidx, sidx2, zeros, out, gi0, gi1, si0, si1, rows, acc,
            *sems):
        c = lax.axis_index("c")
        s = lax.axis_index("s")
        sg = sems[:NSLOT]
        ss = sems[NSLOT:2 * NSLOT]
        sgi = sems[2 * NSLOT:2 * NSLOT + 2]
        ssi = sems[2 * NSLOT + 2:]
        gi = (gi0, gi1)
        si = (si0, si1)

        def gi_src(p):
            return gidx.at[
                pl.ds(c * ec + s * ec_per + p * (PH * CH), PH * CH)]

        def si_src(p):
            return sidx2.at[
                pl.ds((c * NSUB + s) * n_chunks + p * PH, PH)]

        pltpu.async_copy(gi_src(0), gi0, sgi[0])
        pltpu.async_copy(si_src(0), si0, ssi[0])
        pltpu.async_copy(gi_src(1), gi1, sgi[1])
        pltpu.async_copy(si_src(1), si1, ssi[1])
        pltpu.sync_copy(zeros, acc.at[pl.ds(s * NPS, NPS)])
        plsc.subcore_barrier()

        @pl.loop(0, n_phases // 2)
        def _(pp):
            for half in (0, 1):
                p = pp * 2 + half
                pltpu.make_async_copy(gi_src(p), gi[half], sgi[half]).wait()
                pltpu.make_async_copy(si_src(p), si[half], ssi[half]).wait()
                gi_v = gi[half]
                si_v = si[half]

                hg = [None] * NSLOT
                hs = [None] * NSLOT
                for i in range(PH):
                    b = i % NSLOT
                    if hs[b] is not None:
                        hs[b].wait()
                    hg[b] = pltpu.async_copy(
                        table.at[gi_v.at[pl.ds(i * CH, CH)]], rows.at[b],
                        sg[b])
                    if i >= 3:
                        b2 = (i - 3) % NSLOT
                        hg[b2].wait()
                        hs[b2] = pltpu.async_copy(
                            rows.at[b2], acc.at[si_v.at[i - 3]], ss[b2],
                            add=True)
                for i in (PH - 3, PH - 2, PH - 1):
                    b = i % NSLOT
                    hg[b].wait()
                    hs[b] = pltpu.async_copy(
                        rows.at[b], acc.at[si_v.at[i]], ss[b], add=True)
                for b in range(NSLOT):
                    if hs[b] is not None:
                        hs[b].wait()

                @pl.when(p + 2 < n_phases)
                def _():
                    pltpu.async_copy(gi_src(p + 2), gi[half], sgi[half])
                    pltpu.async_copy(si_src(p + 2), si[half], ssi[half])

        plsc.subcore_barrier()
        pltpu.sync_copy(acc.at[pl.ds(s * NPS, NPS)],
                        out.at[pl.ds(c * NP + s * NPS, NPS)])

    return seg


DW = 128


def _make_degree():
    ec_per = E_PAD // NSUB
    n_chunks = ec_per // CH
    n_phases = n_chunks // PH
    mesh = plsc.VectorSubcoreMesh(core_axis_name="c", subcore_axis_name="s")

    @functools.partial(
        pl.kernel,
        out_type=jax.ShapeDtypeStruct((2 * NP, DW), jnp.float32),
        mesh=mesh,
        scratch_types=[
            pltpu.VMEM((PH, CH), jnp.int32),
            pltpu.VMEM((CH, DW), jnp.float32),
            pltpu.VMEM_SHARED((NP, DW), jnp.float32),
        ] + [pltpu.SemaphoreType.DMA] * NSLOT,
    )
    def deg(cidx2, ones_hbm, zeros_d, out, si_v, ones_v, acc, *ss):
        c = lax.axis_index("c")
        s = lax.axis_index("s")
        pltpu.sync_copy(zeros_d, acc.at[pl.ds(s * NPS, NPS)])
        pltpu.sync_copy(ones_hbm, ones_v)
        plsc.subcore_barrier()

        @pl.loop(0, n_phases)
        def _(p):
            pltpu.sync_copy(
                cidx2.at[pl.ds((c * NSUB + s) * n_chunks + p * PH, PH)], si_v)

            hs = [None] * NSLOT
            for i in range(PH):
                b = i % NSLOT
                if hs[b] is not None:
                    hs[b].wait()
                hs[b] = pltpu.async_copy(
                    ones_v, acc.at[si_v.at[i]], ss[b], add=True)
            for b in range(NSLOT):
                if hs[b] is not None:
                    hs[b].wait()

        plsc.subcore_barrier()
        pltpu.sync_copy(acc.at[pl.ds(s * NPS, NPS)],
                        out.at[pl.ds(c * NP + s * NPS, NPS)])

    return deg


_CACHE = {}


def _seg_a(*args):
    if "a" not in _CACHE:
        _CACHE["a"] = _make_segsum(E_PAD // 2)
    return _CACHE["a"](*args)


def _seg_c(*args):
    if "c" not in _CACHE:
        _CACHE["c"] = _make_segsum(E_PAD)
    return _CACHE["c"](*args)


def _degree(*args):
    if "d" not in _CACHE:
        _CACHE["d"] = _make_degree()
    return _CACHE["d"](*args)


def _enc_body(x, we, be, cnt, h_ref, ht_ref):
    h = jnp.maximum(
        jnp.dot(x[...], we[...], preferred_element_type=jnp.float32)
        + be[...], 0.0)
    dinv = lax.rsqrt(cnt[...][1][:, 0:1] + 1.0)
    h_ref[...] = h
    ht_ref[...] = h * dinv


def _conv_body(p, h, cnt, wc, bc, wg, bg, hn_ref, g_ref):
    pp = p[...]
    dinv = lax.rsqrt(cnt[...][1][:, 0:1] + 1.0)
    a = dinv * (pp[0] + pp[1]) + (dinv * dinv) * h[...]
    hn = jnp.maximum(
        jnp.dot(a, wc[...], preferred_element_type=jnp.float32) + bc[...], 0.0)
    hg = jnp.maximum(
        jnp.dot(a, wg[...], preferred_element_type=jnp.float32) + bg[...], 0.0)
    hn_ref[...] = hn
    g_ref[...] = jnp.stack([hg, hg * hg])


def _gate_body(t, g, h, hn, cnt, ho_ref, hto_ref):
    tt = t[...]
    gg = g[...]
    cc = cnt[...]
    outdeg = cc[0][:, 0:1]
    dinv = lax.rsqrt(cc[1][:, 0:1] + 1.0)
    invc = 1.0 / jnp.maximum(outdeg, 1.0)
    s = outdeg * gg[1] - 2.0 * gg[0] * tt[0] + tt[1]
    tau = jnp.tanh(s * invc)
    ho = h[...] + tau * (hn[...] - h[...])
    ho_ref[...] = ho
    hto_ref[...] = ho * dinv


def _dec_body(h, wd, bd, o_ref):
    o_ref[...] = jnp.maximum(
        jnp.dot(h[...], wd[...], preferred_element_type=jnp.float32)
        + bd[...], 0.0)


_ROW = pl.BlockSpec((BN, D), lambda i: (i, 0))
_MAT = pl.BlockSpec((D, D), lambda i: (0, 0))
_VEC = pl.BlockSpec((1, D), lambda i: (0, 0))
_CNT = pl.BlockSpec((2, BN, 16), lambda i: (0, i, 0))
_ROW2 = pl.BlockSpec((2, BN, D), lambda i: (0, i, 0))
_F = jax.ShapeDtypeStruct((N, D), jnp.float32)
_F2 = jax.ShapeDtypeStruct((2, N, D), jnp.float32)


def _enc(x, we, be, cnt2):
    return pl.pallas_call(
        _enc_body, grid=(N // BN,),
        in_specs=[_ROW, _MAT, _VEC, _CNT],
        out_specs=[_ROW, _ROW], out_shape=[_F, _F],
    )(x, we, be, cnt2)


def _conv(p, h, cnt2, wc, bc, wg, bg):
    return pl.pallas_call(
        _conv_body, grid=(N // BN,),
        in_specs=[_ROW2, _ROW, _CNT, _MAT, _VEC, _MAT, _VEC],
        out_specs=[_ROW, _ROW2], out_shape=[_F, _F2],
    )(p, h, cnt2, wc, bc, wg, bg)


def _gate(t, g, h, hn, cnt2):
    return pl.pallas_call(
        _gate_body, grid=(N // BN,),
        in_specs=[_ROW2, _ROW2, _ROW, _ROW, _CNT],
        out_specs=[_ROW, _ROW], out_shape=[_F, _F],
    )(t, g, h, hn, cnt2)


def _dec(h, wd, bd):
    return pl.pallas_call(
        _dec_body, grid=(N // BN,),
        in_specs=[_ROW, _MAT, _VEC],
        out_specs=_ROW, out_shape=_F,
    )(h, wd, bd)


def kernel(X, edge_index, W_enc, b_enc, W_conv, b_conv, W_gg, b_gg,
           W_dec, b_dec):
    src0 = edge_index[0]
    dst0 = edge_index[1]
    pad_junk = jnp.full((PAD,), N, jnp.int32)
    pad_zero = jnp.zeros((PAD,), jnp.int32)
    src_s = jnp.concatenate([src0, pad_junk])
    dst_s = jnp.concatenate([dst0, pad_junk])
    src_g = jnp.concatenate([src0, pad_zero])
    dst_g = jnp.concatenate([dst0, pad_zero])

    zeros_d = jnp.zeros((NPS, D), jnp.float32)
    zeros_w = jnp.zeros((NPS, DW), jnp.float32)
    ones_w = jnp.ones((CH, DW), jnp.float32)

    cidx2 = jnp.concatenate([src_s, dst_s]).reshape(-1, CH)
    cnt2 = _degree(cidx2, ones_w, zeros_w).reshape(2, NP, DW)[:, :N, :16]

    be = b_enc.reshape(1, D)
    bc = b_conv.reshape(1, D)
    bg = b_gg.reshape(1, D)
    bd = b_dec.reshape(1, D)

    H, Ht = _enc(X, W_enc, be, cnt2)

    sidx_a = dst_s.reshape(-1, CH)
    gidx_c = jnp.concatenate([dst_g, dst_g + N])
    sidx_c = jnp.concatenate([src_s, src_s]).reshape(-1, CH)

    for _ in range(4):
        Pf = _seg_a(Ht, src_g, sidx_a, zeros_d)
        P = Pf.reshape(2, NP, D)[:, :N]
        Hn, G = _conv(P, H, cnt2, W_conv, bc, W_gg, bg)
        Tf = _seg_c(G.reshape(2 * N, D), gidx_c, sidx_c, zeros_d)
        T = Tf.reshape(2, NP, D)[:, :N]
        H, Ht = _gate(T, G, H, Hn, cnt2)

    return _dec(H, W_dec, bd)


# speedup vs baseline: 6.0643x; 1.0021x over previous
import functools

import jax
import jax.numpy as jnp
from jax import lax
from jax.experimental import pallas as pl
from jax.experimental.pallas import tpu as pltpu
from jax.experimental.pallas import tpu_sc as plsc

N = 10000
E = 320000
D = 128
NSUB = 16
NCORE = 2
CH = 64
E_PAD = 327680
PAD = E_PAD - E
NP = 10240
NPS = NP // NSUB
BN = 2000
NSLOT = 5
PH = 16


def _make_segsum(ec):
    ec_per = ec // NSUB
    n_chunks = ec_per // CH
    n_phases = n_chunks // PH
    mesh = plsc.VectorSubcoreMesh(core_axis_name="c", subcore_axis_name="s")

    @functools.partial(
        pl.kernel,
        out_type=jax.ShapeDtypeStruct((2 * NP, D), jnp.float32),
        mesh=mesh,
        scratch_types=[
            pltpu.VMEM((PH * CH,), jnp.int32),
            pltpu.VMEM((PH * CH,), jnp.int32),
            pltpu.VMEM((PH, CH), jnp.int32),
            pltpu.VMEM((PH, CH), jnp.int32),
            pltpu.VMEM((NSLOT, CH, D), jnp.float32),
            pltpu.VMEM_SHARED((NP, D), jnp.float32),
        ] + [pltpu.SemaphoreType.DMA] * (2 * NSLOT + 4),
    )
    def seg(table, gidx, sidx2, zeros, out, gi0, gi1, si0, si1, rows, acc,
            *sems):
        c = lax.axis_index("c")
        s = lax.axis_index("s")
        sg = sems[:NSLOT]
        ss = sems[NSLOT:2 * NSLOT]
        sgi = sems[2 * NSLOT:2 * NSLOT + 2]
        ssi = sems[2 * NSLOT + 2:]
        gi = (gi0, gi1)
        si = (si0, si1)

        def gi_src(p):
            return gidx.at[
                pl.ds(c * ec + s * ec_per + p * (PH * CH), PH * CH)]

        def si_src(p):
            return sidx2.at[
                pl.ds((c * NSUB + s) * n_chunks + p * PH, PH)]

        pltpu.async_copy(gi_src(0), gi0, sgi[0])
        pltpu.async_copy(si_src(0), si0, ssi[0])
        pltpu.async_copy(gi_src(1), gi1, sgi[1])
        pltpu.async_copy(si_src(1), si1, ssi[1])
        pltpu.sync_copy(zeros, acc.at[pl.ds(s * NPS, NPS)])
        plsc.subcore_barrier()

        @pl.loop(0, n_phases // 2)
        def _(pp):
            for half in (0, 1):
                p = pp * 2 + half
                pltpu.make_async_copy(gi_src(p), gi[half], sgi[half]).wait()
                pltpu.make_async_copy(si_src(p), si[half], ssi[half]).wait()
                gi_v = gi[half]
                si_v = si[half]

                hg = [None] * NSLOT
                hs = [None] * NSLOT
                for i in range(PH):
                    b = i % NSLOT
                    if hs[b] is not None:
                        hs[b].wait()
                    hg[b] = pltpu.async_copy(
                        table.at[gi_v.at[pl.ds(i * CH, CH)]], rows.at[b],
                        sg[b])
                    if i >= 3:
                        b2 = (i - 3) % NSLOT
                        hg[b2].wait()
                        hs[b2] = pltpu.async_copy(
                            rows.at[b2], acc.at[si_v.at[i - 3]], ss[b2],
                            add=True)
                for i in (PH - 3, PH - 2, PH - 1):
                    b = i % NSLOT
                    hg[b].wait()
                    hs[b] = pltpu.async_copy(
                        rows.at[b], acc.at[si_v.at[i]], ss[b], add=True)
                for b in range(NSLOT):
                    if hs[b] is not None:
                        hs[b].wait()

                @pl.when(p + 2 < n_phases)
                def _():
                    pltpu.async_copy(gi_src(p + 2), gi[half], sgi[half])
                    pltpu.async_copy(si_src(p + 2), si[half], ssi[half])

        plsc.subcore_barrier()
        pltpu.sync_copy(acc.at[pl.ds(s * NPS, NPS)],
                        out.at[pl.ds(c * NP + s * NPS, NPS)])

    return seg


DW = 128


def _make_degree():
    ec_per = E_PAD // NSUB
    n_chunks = ec_per // CH
    n_phases = n_chunks // PH
    mesh = plsc.VectorSubcoreMesh(core_axis_name="c", subcore_axis_name="s")

    @functools.partial(
        pl.kernel,
        out_type=jax.ShapeDtypeStruct((2 * NP, DW), jnp.float32),
        mesh=mesh,
        scratch_types=[
            pltpu.VMEM((PH, CH), jnp.int32),
            pltpu.VMEM((PH, CH), jnp.int32),
            pltpu.VMEM((CH, DW), jnp.float32),
            pltpu.VMEM_SHARED((NP, DW), jnp.float32),
        ] + [pltpu.SemaphoreType.DMA] * (NSLOT + 2),
    )
    def deg(cidx2, ones_hbm, zeros_d, out, si0, si1, ones_v, acc, *sems):
        c = lax.axis_index("c")
        s = lax.axis_index("s")
        ss = sems[:NSLOT]
        ssi = sems[NSLOT:]
        si = (si0, si1)

        def si_src(p):
            return cidx2.at[pl.ds((c * NSUB + s) * n_chunks + p * PH, PH)]

        pltpu.async_copy(si_src(0), si0, ssi[0])
        pltpu.async_copy(si_src(1), si1, ssi[1])
        pltpu.sync_copy(zeros_d, acc.at[pl.ds(s * NPS, NPS)])
        pltpu.sync_copy(ones_hbm, ones_v)
        plsc.subcore_barrier()

        @pl.loop(0, n_phases // 2)
        def _(pp):
            for half in (0, 1):
                p = pp * 2 + half
                pltpu.make_async_copy(si_src(p), si[half], ssi[half]).wait()
                si_v = si[half]

                hs = [None] * NSLOT
                for i in range(PH):
                    b = i % NSLOT
                    if hs[b] is not None:
                        hs[b].wait()
                    hs[b] = pltpu.async_copy(
                        ones_v, acc.at[si_v.at[i]], ss[b], add=True)
                for b in range(NSLOT):
                    if hs[b] is not None:
                        hs[b].wait()

                @pl.when(p + 2 < n_phases)
                def _():
                    pltpu.async_copy(si_src(p + 2), si[half], ssi[half])

        plsc.subcore_barrier()
        pltpu.sync_copy(acc.at[pl.ds(s * NPS, NPS)],
                        out.at[pl.ds(c * NP + s * NPS, NPS)])

    return deg


_CACHE = {}


def _seg_a(*args):
    if "a" not in _CACHE:
        _CACHE["a"] = _make_segsum(E_PAD // 2)
    return _CACHE["a"](*args)


def _seg_c(*args):
    if "c" not in _CACHE:
        _CACHE["c"] = _make_segsum(E_PAD)
    return _CACHE["c"](*args)


def _degree(*args):
    if "d" not in _CACHE:
        _CACHE["d"] = _make_degree()
    return _CACHE["d"](*args)


def _enc_body(x, we, be, cnt, h_ref, ht_ref):
    h = jnp.maximum(
        jnp.dot(x[...], we[...], preferred_element_type=jnp.float32)
        + be[...], 0.0)
    dinv = lax.rsqrt(cnt[...][1][:, 0:1] + 1.0)
    h_ref[...] = h
    ht_ref[...] = h * dinv


def _conv_body(p, h, cnt, wc, bc, wg, bg, hn_ref, g_ref):
    pp = p[...]
    dinv = lax.rsqrt(cnt[...][1][:, 0:1] + 1.0)
    a = dinv * (pp[0] + pp[1]) + (dinv * dinv) * h[...]
    hn = jnp.maximum(
        jnp.dot(a, wc[...], preferred_element_type=jnp.float32) + bc[...], 0.0)
    hg = jnp.maximum(
        jnp.dot(a, wg[...], preferred_element_type=jnp.float32) + bg[...], 0.0)
    hn_ref[...] = hn
    g_ref[...] = jnp.stack([hg, hg * hg])


def _gate_body(t, g, h, hn, cnt, ho_ref, hto_ref):
    tt = t[...]
    gg = g[...]
    cc = cnt[...]
    outdeg = cc[0][:, 0:1]
    dinv = lax.rsqrt(cc[1][:, 0:1] + 1.0)
    invc = 1.0 / jnp.maximum(outdeg, 1.0)
    s = outdeg * gg[1] - 2.0 * gg[0] * tt[0] + tt[1]
    tau = jnp.tanh(s * invc)
    ho = h[...] + tau * (hn[...] - h[...])
    ho_ref[...] = ho
    hto_ref[...] = ho * dinv


def _dec_body(h, wd, bd, o_ref):
    o_ref[...] = jnp.maximum(
        jnp.dot(h[...], wd[...], preferred_element_type=jnp.float32)
        + bd[...], 0.0)


_ROW = pl.BlockSpec((BN, D), lambda i: (i, 0))
_MAT = pl.BlockSpec((D, D), lambda i: (0, 0))
_VEC = pl.BlockSpec((1, D), lambda i: (0, 0))
_CNT = pl.BlockSpec((2, BN, 16), lambda i: (0, i, 0))
_ROW2 = pl.BlockSpec((2, BN, D), lambda i: (0, i, 0))
_F = jax.ShapeDtypeStruct((N, D), jnp.float32)
_F2 = jax.ShapeDtypeStruct((2, N, D), jnp.float32)


def _enc(x, we, be, cnt2):
    return pl.pallas_call(
        _enc_body, grid=(N // BN,),
        in_specs=[_ROW, _MAT, _VEC, _CNT],
        out_specs=[_ROW, _ROW], out_shape=[_F, _F],
    )(x, we, be, cnt2)


def _conv(p, h, cnt2, wc, bc, wg, bg):
    return pl.pallas_call(
        _conv_body, grid=(N // BN,),
        in_specs=[_ROW2, _ROW, _CNT, _MAT, _VEC, _MAT, _VEC],
        out_specs=[_ROW, _ROW2], out_shape=[_F, _F2],
    )(p, h, cnt2, wc, bc, wg, bg)


def _gate(t, g, h, hn, cnt2):
    return pl.pallas_call(
        _gate_body, grid=(N // BN,),
        in_specs=[_ROW2, _ROW2, _ROW, _ROW, _CNT],
        out_specs=[_ROW, _ROW], out_shape=[_F, _F],
    )(t, g, h, hn, cnt2)


def _dec(h, wd, bd):
    return pl.pallas_call(
        _dec_body, grid=(N // BN,),
        in_specs=[_ROW, _MAT, _VEC],
        out_specs=_ROW, out_shape=_F,
    )(h, wd, bd)


def kernel(X, edge_index, W_enc, b_enc, W_conv, b_conv, W_gg, b_gg,
           W_dec, b_dec):
    src0 = edge_index[0]
    dst0 = edge_index[1]
    pad_junk = jnp.full((PAD,), N, jnp.int32)
    pad_zero = jnp.zeros((PAD,), jnp.int32)
    src_s = jnp.concatenate([src0, pad_junk])
    dst_s = jnp.concatenate([dst0, pad_junk])
    src_g = jnp.concatenate([src0, pad_zero])
    dst_g = jnp.concatenate([dst0, pad_zero])

    zeros_d = jnp.zeros((NPS, D), jnp.float32)
    zeros_w = jnp.zeros((NPS, DW), jnp.float32)
    ones_w = jnp.ones((CH, DW), jnp.float32)

    cidx2 = jnp.concatenate([src_s, dst_s]).reshape(-1, CH)
    cnt2 = _degree(cidx2, ones_w, zeros_w).reshape(2, NP, DW)[:, :N, :16]

    be = b_enc.reshape(1, D)
    bc = b_conv.reshape(1, D)
    bg = b_gg.reshape(1, D)
    bd = b_dec.reshape(1, D)

    H, Ht = _enc(X, W_enc, be, cnt2)

    sidx_a = dst_s.reshape(-1, CH)
    gidx_c = jnp.concatenate([dst_g, dst_g + N])
    sidx_c = jnp.concatenate([src_s, src_s]).reshape(-1, CH)

    for _ in range(4):
        Pf = _seg_a(Ht, src_g, sidx_a, zeros_d)
        P = Pf.reshape(2, NP, D)[:, :N]
        Hn, G = _conv(P, H, cnt2, W_conv, bc, W_gg, bg)
        Tf = _seg_c(G.reshape(2 * N, D), gidx_c, sidx_c, zeros_d)
        T = Tf.reshape(2, NP, D)[:, :N]
        H, Ht = _gate(T, G, H, Hn, cnt2)

    return _dec(H, W_dec, bd)
